# baseline (speedup 1.0000x reference)
import functools

import jax
import jax.numpy as jnp
from jax import lax
from jax.experimental import pallas as pl
from jax.experimental.pallas import tpu as pltpu
from jax.experimental.pallas import tpu_sc as plsc

N = 10000
E = 320000
D = 128
H = 128
OUT = 40

NC, NS = 2, 16
NTILE = NC * NS
CHUNK = 128
NBUF = 2
_EQ = NTILE * CHUNK * NBUF
EPAD = ((E + _EQ - 1) // _EQ) * _EQ
CPP = EPAD // (NS * CHUNK)
C0 = 128
C1 = CPP - C0
PKB = 80
NSTAGE = (max(C0, C1) + PKB - 1) // PKB
NPAD = 10240
RPT = NPAD // NS
DEGW = 128

BR = 1280
GRID = NPAD // BR

_mesh = plsc.VectorSubcoreMesh(
    core_axis_name="c", subcore_axis_name="s", num_cores=NC, num_subcores=NS
)


@functools.partial(
    pl.kernel,
    out_type=jax.ShapeDtypeStruct((NC, NPAD, H), jnp.float32),
    mesh=_mesh,
    scratch_types=[
        pltpu.VMEM((PKB, CHUNK), jnp.int32),
        [pltpu.VMEM((CHUNK,), jnp.int32)] * NBUF,
        [pltpu.VMEM((CHUNK,), jnp.int32)] * NBUF,
        [pltpu.VMEM((CHUNK, H), jnp.float32)] * NBUF,
        [pltpu.SemaphoreType.DMA] * NBUF,
        pltpu.VMEM_SHARED((NPAD, H), jnp.float32),
    ],
)
def _sc_aggregate(zs_hbm, pk_hbm, agg_hbm, pkv, idx_s, idx_d, rows, sems, acc):
    c = lax.axis_index("c")
    s = lax.axis_index("s")
    r0 = s * RPT
    pltpu.sync_copy(zs_hbm.at[pl.ds(r0, RPT)], acc.at[pl.ds(r0, RPT)])
    plsc.subcore_barrier()

    def unpack(i, j):
        for v in range(CHUNK // 16):
            p = pkv[i, pl.ds(v * 16, 16)]
            idx_s[j][pl.ds(v * 16, 16)] = lax.bitwise_and(p, 0xFFFF)
            idx_d[j][pl.ds(v * 16, 16)] = lax.shift_right_logical(p, 16)

    def run_stages(gbase, stage_counts):
        for st, cnt in enumerate(stage_counts):
            pltpu.sync_copy(pk_hbm.at[pl.ds(gbase + st * PKB, PKB)], pkv)
            for j in range(NBUF):
                unpack(j, j)
                pltpu.async_copy(zs_hbm.at[idx_s[j]], rows[j], sems[j])

            def group(g, carry):
                for j in range(NBUF):
                    i = g * NBUF + j
                    pltpu.make_async_copy(zs_hbm.at[idx_s[j]], rows[j], sems[j]).wait()
                    pltpu.sync_copy(rows[j], acc.at[idx_d[j]], add=True)

                    @pl.when(i + NBUF < cnt)
                    def _():
                        unpack(i + NBUF, j)
                        pltpu.async_copy(zs_hbm.at[idx_s[j]], rows[j], sems[j])

                return carry

            lax.fori_loop(0, cnt // NBUF, group, 0)

    @pl.when(c == 0)
    def _core0():
        run_stages(s * CPP, [PKB] * (C0 // PKB) + ([C0 % PKB] if C0 % PKB else []))

    @pl.when(c == 1)
    def _core1():
        run_stages(s * CPP + C0, [PKB] * (C1 // PKB) + ([C1 % PKB] if C1 % PKB else []))

    plsc.subcore_barrier()
    pltpu.sync_copy(acc.at[pl.ds(r0, RPT)], agg_hbm.at[c, pl.ds(r0, RPT)])


def _row_spec():
    return pl.BlockSpec((BR, H), lambda i: (i, 0))


def _full_spec(shape):
    return pl.BlockSpec(shape, lambda i: tuple(0 for _ in shape))


def _layer1_body(x_ref, d0_ref, d1_ref, w_ref, zs_ref, dinv_ref):
    i = pl.program_id(0)
    deg = d0_ref[:, 0:1] + d1_ref[:, 0:1] - 1.0
    dinv = jnp.broadcast_to(lax.rsqrt(deg), (BR, H))
    rid = lax.broadcasted_iota(jnp.int32, (BR, H), 0) + i * BR
    dinv = jnp.where(rid < N, dinv, 0.0)
    dinv_ref[...] = dinv
    zs_ref[...] = jnp.dot(dinv * x_ref[...], w_ref[...],
                          preferred_element_type=jnp.float32)


def _tc_layer1(xp, d0, d1, w):
    return pl.pallas_call(
        _layer1_body,
        grid=(GRID,),
        in_specs=[
            _row_spec(),
            pl.BlockSpec((BR, DEGW), lambda i: (i, 0)),
            pl.BlockSpec((BR, DEGW), lambda i: (i, 0)),
            _full_spec((D, H)),
        ],
        out_specs=[_row_spec(), _row_spec()],
        out_shape=[
            jax.ShapeDtypeStruct((NPAD, H), jnp.float32),
            jax.ShapeDtypeStruct((NPAD, H), jnp.float32),
        ],
    )(xp, d0, d1, w)


def _combine_body(a0_ref, a1_ref, zs_ref, dinv_ref, b_ref, w_ref, out_ref):
    dinv = dinv_ref[...]
    h = jnp.maximum(dinv * (a0_ref[...] + a1_ref[...] - zs_ref[...]) + b_ref[...], 0.0)
    out_ref[...] = jnp.dot(dinv * h, w_ref[...], preferred_element_type=jnp.float32)


def _tc_combine(a0, a1, zs, dinv2d, b, w):
    return pl.pallas_call(
        _combine_body,
        grid=(GRID,),
        in_specs=[
            _row_spec(), _row_spec(), _row_spec(), _row_spec(),
            _full_spec((1, H)), _full_spec((H, H)),
        ],
        out_specs=_row_spec(),
        out_shape=jax.ShapeDtypeStruct((NPAD, H), jnp.float32),
    )(a0, a1, zs, dinv2d, b, w)


def _final_body(a0_ref, a1_ref, zs_ref, dinv_ref, b_ref, wr_ref, br_ref, out_ref):
    dinv = dinv_ref[...]
    h = jnp.maximum(dinv * (a0_ref[...] + a1_ref[...] - zs_ref[...]) + b_ref[...], 0.0)
    logits = jnp.dot(h, wr_ref[...], preferred_element_type=jnp.float32) + br_ref[...]
    m = jnp.max(logits, axis=1, keepdims=True)
    e = jnp.exp(logits - m)
    out_ref[...] = e / jnp.sum(e, axis=1, keepdims=True)


def _tc_final(a0, a1, zs, dinv2d, b, wr_pad, br_pad):
    return pl.pallas_call(
        _final_body,
        grid=(GRID,),
        in_specs=[
            _row_spec(), _row_spec(), _row_spec(), _row_spec(),
            _full_spec((1, H)), _full_spec((H, H)), _full_spec((1, H)),
        ],
        out_specs=_row_spec(),
        out_shape=jax.ShapeDtypeStruct((NPAD, H), jnp.float32),
    )(a0, a1, zs, dinv2d, b, wr_pad, br_pad)


def kernel(x, edge_index, W1, b1, W2, b2, W3, b3, Wr, br):
    f32 = jnp.float32
    xp = jnp.zeros((NPAD, D), f32).at[:N].set(x)
    pk = jnp.full((EPAD + PKB * CHUNK,), N | (N << 16), jnp.int32)
    pk = pk.at[:E].set(edge_index[0] | (edge_index[1] << 16)).reshape(-1, CHUNK)
    ones_mat = jnp.ones((NPAD, H), f32)
    wr_pad = jnp.zeros((H, H), f32).at[:, :OUT].set(Wr)
    br_pad = jnp.full((1, H), -1e30, f32).at[0, :OUT].set(br)

    d2 = _sc_aggregate(ones_mat, pk)
    zs1, dinv2d = _tc_layer1(xp, d2[0], d2[1], W1)
    a1 = _sc_aggregate(zs1, pk)
    zs2 = _tc_combine(a1[0], a1[1], zs1, dinv2d, b1.reshape(1, H), W2)
    a2 = _sc_aggregate(zs2, pk)
    zs3 = _tc_combine(a2[0], a2[1], zs2, dinv2d, b2.reshape(1, H), W3)
    a3 = _sc_aggregate(zs3, pk)
    probs = _tc_final(a3[0], a3[1], zs3, dinv2d, b3.reshape(1, H), wr_pad, br_pad)
    return probs[:N, :OUT]


# speedup vs baseline: 1.0466x; 1.0466x over previous
import functools

import jax
import jax.numpy as jnp
from jax import lax
from jax.experimental import pallas as pl
from jax.experimental.pallas import tpu as pltpu
from jax.experimental.pallas import tpu_sc as plsc

N = 10000
E = 320000
D = 128
H = 128
OUT = 40

NC, NS = 2, 16
NTILE = NC * NS
CHUNK = 128
NBUF = 2
_EQ = NTILE * CHUNK * NBUF
EPAD = ((E + _EQ - 1) // _EQ) * _EQ
CPP = EPAD // (NS * CHUNK)
C0 = 120
C1 = CPP - C0
PKB = C0
NPAD = 10240
RPT = NPAD // NS
DEGW = 128

BR = 1280
GRID = NPAD // BR

_mesh = plsc.VectorSubcoreMesh(
    core_axis_name="c", subcore_axis_name="s", num_cores=NC, num_subcores=NS
)


@functools.partial(
    pl.kernel,
    out_type=jax.ShapeDtypeStruct((NC, NPAD, H), jnp.float32),
    mesh=_mesh,
    scratch_types=[
        pltpu.VMEM((PKB, CHUNK), jnp.int32),
        [pltpu.VMEM((CHUNK,), jnp.int32)] * NBUF,
        [pltpu.VMEM((CHUNK,), jnp.int32)] * NBUF,
        [pltpu.VMEM((CHUNK, H), jnp.float32)] * NBUF,
        [pltpu.SemaphoreType.DMA] * NBUF,
        pltpu.VMEM_SHARED((NPAD, H), jnp.float32),
    ],
)
def _sc_aggregate(zs_hbm, pk_hbm, agg_hbm, pkv, idx_s, idx_d, rows, sems, acc):
    c = lax.axis_index("c")
    s = lax.axis_index("s")
    r0 = s * RPT
    pltpu.sync_copy(zs_hbm.at[pl.ds(r0, RPT)], acc.at[pl.ds(r0, RPT)])
    plsc.subcore_barrier()

    def unpack(i, j):
        for v in range(CHUNK // 16):
            p = pkv[i, pl.ds(v * 16, 16)]
            idx_s[j][pl.ds(v * 16, 16)] = lax.bitwise_and(p, 0xFFFF)
            idx_d[j][pl.ds(v * 16, 16)] = lax.shift_right_logical(p, 16)

    def run_edges(gbase, cnt):
        pltpu.sync_copy(pk_hbm.at[pl.ds(gbase, PKB)], pkv)
        for j in range(NBUF):
            unpack(j, j)
            pltpu.async_copy(zs_hbm.at[idx_s[j]], rows[j], sems[j])

        def group(g, carry):
            for j in range(NBUF):
                i = g * NBUF + j
                pltpu.make_async_copy(zs_hbm.at[idx_s[j]], rows[j], sems[j]).wait()
                pltpu.sync_copy(rows[j], acc.at[idx_d[j]], add=True)

                @pl.when(i + NBUF < cnt)
                def _():
                    unpack(i + NBUF, j)
                    pltpu.async_copy(zs_hbm.at[idx_s[j]], rows[j], sems[j])

            return carry

        lax.fori_loop(0, cnt // NBUF, group, 0)

    @pl.when(c == 0)
    def _core0():
        run_edges(s * CPP, C0)

    @pl.when(c == 1)
    def _core1():
        run_edges(s * CPP + C0, C1)

    plsc.subcore_barrier()
    pltpu.sync_copy(acc.at[pl.ds(r0, RPT)], agg_hbm.at[c, pl.ds(r0, RPT)])


def _row_spec():
    return pl.BlockSpec((BR, H), lambda i: (i, 0))


def _full_spec(shape):
    return pl.BlockSpec(shape, lambda i: tuple(0 for _ in shape))


def _layer1_body(x_ref, d0_ref, d1_ref, w_ref, zs_ref, dinv_ref):
    i = pl.program_id(0)
    deg = d0_ref[:, 0:1] + d1_ref[:, 0:1] - 1.0
    dinv = jnp.broadcast_to(lax.rsqrt(deg), (BR, H))
    rid = lax.broadcasted_iota(jnp.int32, (BR, H), 0) + i * BR
    dinv = jnp.where(rid < N, dinv, 0.0)
    dinv_ref[...] = dinv
    zs_ref[...] = jnp.dot(dinv * x_ref[...], w_ref[...],
                          preferred_element_type=jnp.float32)


def _tc_layer1(xp, d0, d1, w):
    return pl.pallas_call(
        _layer1_body,
        grid=(GRID,),
        in_specs=[
            _row_spec(),
            pl.BlockSpec((BR, DEGW), lambda i: (i, 0)),
            pl.BlockSpec((BR, DEGW), lambda i: (i, 0)),
            _full_spec((D, H)),
        ],
        out_specs=[_row_spec(), _row_spec()],
        out_shape=[
            jax.ShapeDtypeStruct((NPAD, H), jnp.float32),
            jax.ShapeDtypeStruct((NPAD, H), jnp.float32),
        ],
    )(xp, d0, d1, w)


def _combine_body(a0_ref, a1_ref, zs_ref, dinv_ref, b_ref, w_ref, out_ref):
    dinv = dinv_ref[...]
    h = jnp.maximum(dinv * (a0_ref[...] + a1_ref[...] - zs_ref[...]) + b_ref[...], 0.0)
    out_ref[...] = jnp.dot(dinv * h, w_ref[...], preferred_element_type=jnp.float32)


def _tc_combine(a0, a1, zs, dinv2d, b, w):
    return pl.pallas_call(
        _combine_body,
        grid=(GRID,),
        in_specs=[
            _row_spec(), _row_spec(), _row_spec(), _row_spec(),
            _full_spec((1, H)), _full_spec((H, H)),
        ],
        out_specs=_row_spec(),
        out_shape=jax.ShapeDtypeStruct((NPAD, H), jnp.float32),
    )(a0, a1, zs, dinv2d, b, w)


def _final_body(a0_ref, a1_ref, zs_ref, dinv_ref, b_ref, wr_ref, br_ref, out_ref):
    dinv = dinv_ref[...]
    h = jnp.maximum(dinv * (a0_ref[...] + a1_ref[...] - zs_ref[...]) + b_ref[...], 0.0)
    logits = jnp.dot(h, wr_ref[...], preferred_element_type=jnp.float32) + br_ref[...]
    m = jnp.max(logits, axis=1, keepdims=True)
    e = jnp.exp(logits - m)
    out_ref[...] = e / jnp.sum(e, axis=1, keepdims=True)


def _tc_final(a0, a1, zs, dinv2d, b, wr_pad, br_pad):
    return pl.pallas_call(
        _final_body,
        grid=(GRID,),
        in_specs=[
            _row_spec(), _row_spec(), _row_spec(), _row_spec(),
            _full_spec((1, H)), _full_spec((H, H)), _full_spec((1, H)),
        ],
        out_specs=_row_spec(),
        out_shape=jax.ShapeDtypeStruct((NPAD, H), jnp.float32),
    )(a0, a1, zs, dinv2d, b, wr_pad, br_pad)


def kernel(x, edge_index, W1, b1, W2, b2, W3, b3, Wr, br):
    f32 = jnp.float32
    xp = jnp.zeros((NPAD, D), f32).at[:N].set(x)
    pk = jnp.full((EPAD + PKB * CHUNK,), N | (N << 16), jnp.int32)
    pk = pk.at[:E].set(edge_index[0] | (edge_index[1] << 16)).reshape(-1, CHUNK)
    ones_mat = jnp.ones((NPAD, H), f32)
    wr_pad = jnp.zeros((H, H), f32).at[:, :OUT].set(Wr)
    br_pad = jnp.full((1, H), -1e30, f32).at[0, :OUT].set(br)

    d2 = _sc_aggregate(ones_mat, pk)
    zs1, dinv2d = _tc_layer1(xp, d2[0], d2[1], W1)
    a1 = _sc_aggregate(zs1, pk)
    zs2 = _tc_combine(a1[0], a1[1], zs1, dinv2d, b1.reshape(1, H), W2)
    a2 = _sc_aggregate(zs2, pk)
    zs3 = _tc_combine(a2[0], a2[1], zs2, dinv2d, b2.reshape(1, H), W3)
    a3 = _sc_aggregate(zs3, pk)
    probs = _tc_final(a3[0], a3[1], zs3, dinv2d, b3.reshape(1, H), wr_pad, br_pad)
    return probs[:N, :OUT]


# speedup vs baseline: 1.2646x; 1.2083x over previous
import functools

import jax
import jax.numpy as jnp
from jax import lax
from jax.experimental import pallas as pl
from jax.experimental.pallas import tpu as pltpu
from jax.experimental.pallas import tpu_sc as plsc

N = 10000
E = 320000
D = 128
H = 128
OUT = 40

NC, NS = 2, 16
NTILE = NC * NS
CHUNK = 128
NBUF = 2
_EQ = NTILE * CHUNK * NBUF
EPAD = ((E + _EQ - 1) // _EQ) * _EQ
CPP = EPAD // (NS * CHUNK)
C0 = 120
C1 = CPP - C0
PKB = C0
NPAD = 10240
RPT = NPAD // NS
DEGW = 128

BR = 1280
GRID = NPAD // BR

_mesh = plsc.VectorSubcoreMesh(
    core_axis_name="c", subcore_axis_name="s", num_cores=NC, num_subcores=NS
)


@functools.partial(
    pl.kernel,
    out_type=jax.ShapeDtypeStruct((NC, NPAD, H), jnp.float32),
    mesh=_mesh,
    scratch_types=[
        pltpu.VMEM((PKB, CHUNK), jnp.int32),
        pltpu.VMEM((CHUNK, H), jnp.float32),
        pltpu.VMEM((CHUNK,), jnp.int32),
        pltpu.VMEM_SHARED((NPAD, H), jnp.float32),
    ],
)
def _sc_degree(pk_hbm, ones_hbm, zeros_hbm, deg_hbm, pkv, ones_v, idx_d, hist):
    c = lax.axis_index("c")
    s = lax.axis_index("s")
    r0 = s * RPT
    pltpu.sync_copy(zeros_hbm.at[pl.ds(r0, RPT)], hist.at[pl.ds(r0, RPT)])
    pltpu.sync_copy(ones_hbm.at[pl.ds(0, CHUNK)], ones_v)
    plsc.subcore_barrier()

    def run_edges(gbase, cnt):
        pltpu.sync_copy(pk_hbm.at[pl.ds(gbase, PKB)], pkv)

        def step(i, carry):
            for v in range(CHUNK // 16):
                p = pkv[i, pl.ds(v * 16, 16)]
                idx_d[pl.ds(v * 16, 16)] = lax.shift_right_logical(p, 16)
            pltpu.sync_copy(ones_v, hist.at[idx_d], add=True)
            return carry

        lax.fori_loop(0, cnt, step, 0)

    @pl.when(c == 0)
    def _core0():
        run_edges(s * CPP, C0)

    @pl.when(c == 1)
    def _core1():
        run_edges(s * CPP + C0, C1)

    plsc.subcore_barrier()
    pltpu.sync_copy(hist.at[pl.ds(r0, RPT)], deg_hbm.at[c, pl.ds(r0, RPT)])


@functools.partial(
    pl.kernel,
    out_type=jax.ShapeDtypeStruct((NC, NPAD, H), jnp.float32),
    mesh=_mesh,
    scratch_types=[
        pltpu.VMEM((PKB, CHUNK), jnp.int32),
        [pltpu.VMEM((CHUNK,), jnp.int32)] * NBUF,
        [pltpu.VMEM((CHUNK,), jnp.int32)] * NBUF,
        [pltpu.VMEM((CHUNK, H), jnp.float32)] * NBUF,
        [pltpu.SemaphoreType.DMA] * NBUF,
        pltpu.VMEM_SHARED((NPAD, H), jnp.float32),
    ],
)
def _sc_aggregate(zs_hbm, pk_hbm, agg_hbm, pkv, idx_s, idx_d, rows, sems, acc):
    c = lax.axis_index("c")
    s = lax.axis_index("s")
    r0 = s * RPT
    pltpu.sync_copy(zs_hbm.at[pl.ds(r0, RPT)], acc.at[pl.ds(r0, RPT)])
    plsc.subcore_barrier()

    def unpack(i, j):
        for v in range(CHUNK // 16):
            p = pkv[i, pl.ds(v * 16, 16)]
            idx_s[j][pl.ds(v * 16, 16)] = lax.bitwise_and(p, 0xFFFF)
            idx_d[j][pl.ds(v * 16, 16)] = lax.shift_right_logical(p, 16)

    def run_edges(gbase, cnt):
        pltpu.sync_copy(pk_hbm.at[pl.ds(gbase, PKB)], pkv)
        for j in range(NBUF):
            unpack(j, j)
            pltpu.async_copy(zs_hbm.at[idx_s[j]], rows[j], sems[j])

        def group(g, carry):
            for j in range(NBUF):
                i = g * NBUF + j
                pltpu.make_async_copy(zs_hbm.at[idx_s[j]], rows[j], sems[j]).wait()
                pltpu.sync_copy(rows[j], acc.at[idx_d[j]], add=True)

                @pl.when(i + NBUF < cnt)
                def _():
                    unpack(i + NBUF, j)
                    pltpu.async_copy(zs_hbm.at[idx_s[j]], rows[j], sems[j])

            return carry

        lax.fori_loop(0, cnt // NBUF, group, 0)

    @pl.when(c == 0)
    def _core0():
        run_edges(s * CPP, C0)

    @pl.when(c == 1)
    def _core1():
        run_edges(s * CPP + C0, C1)

    plsc.subcore_barrier()
    pltpu.sync_copy(acc.at[pl.ds(r0, RPT)], agg_hbm.at[c, pl.ds(r0, RPT)])


def _row_spec():
    return pl.BlockSpec((BR, H), lambda i: (i, 0))


def _full_spec(shape):
    return pl.BlockSpec(shape, lambda i: tuple(0 for _ in shape))


def _layer1_body(x_ref, d0_ref, d1_ref, w_ref, zs_ref, dinv_ref):
    i = pl.program_id(0)
    deg = d0_ref[:, 0:1] + d1_ref[:, 0:1] + 1.0
    dinv = jnp.broadcast_to(lax.rsqrt(deg), (BR, H))
    rid = lax.broadcasted_iota(jnp.int32, (BR, H), 0) + i * BR
    dinv = jnp.where(rid < N, dinv, 0.0)
    dinv_ref[...] = dinv
    zs_ref[...] = jnp.dot(dinv * x_ref[...], w_ref[...],
                          preferred_element_type=jnp.float32)


def _tc_layer1(xp, d0, d1, w):
    return pl.pallas_call(
        _layer1_body,
        grid=(GRID,),
        in_specs=[
            _row_spec(),
            pl.BlockSpec((BR, DEGW), lambda i: (i, 0)),
            pl.BlockSpec((BR, DEGW), lambda i: (i, 0)),
            _full_spec((D, H)),
        ],
        out_specs=[_row_spec(), _row_spec()],
        out_shape=[
            jax.ShapeDtypeStruct((NPAD, H), jnp.float32),
            jax.ShapeDtypeStruct((NPAD, H), jnp.float32),
        ],
    )(xp, d0, d1, w)


def _combine_body(a0_ref, a1_ref, zs_ref, dinv_ref, b_ref, w_ref, out_ref):
    dinv = dinv_ref[...]
    h = jnp.maximum(dinv * (a0_ref[...] + a1_ref[...] - zs_ref[...]) + b_ref[...], 0.0)
    out_ref[...] = jnp.dot(dinv * h, w_ref[...], preferred_element_type=jnp.float32)


def _tc_combine(a0, a1, zs, dinv2d, b, w):
    return pl.pallas_call(
        _combine_body,
        grid=(GRID,),
        in_specs=[
            _row_spec(), _row_spec(), _row_spec(), _row_spec(),
            _full_spec((1, H)), _full_spec((H, H)),
        ],
        out_specs=_row_spec(),
        out_shape=jax.ShapeDtypeStruct((NPAD, H), jnp.float32),
    )(a0, a1, zs, dinv2d, b, w)


def _final_body(a0_ref, a1_ref, zs_ref, dinv_ref, b_ref, wr_ref, br_ref, out_ref):
    dinv = dinv_ref[...]
    h = jnp.maximum(dinv * (a0_ref[...] + a1_ref[...] - zs_ref[...]) + b_ref[...], 0.0)
    logits = jnp.dot(h, wr_ref[...], preferred_element_type=jnp.float32) + br_ref[...]
    m = jnp.max(logits, axis=1, keepdims=True)
    e = jnp.exp(logits - m)
    out_ref[...] = e / jnp.sum(e, axis=1, keepdims=True)


def _tc_final(a0, a1, zs, dinv2d, b, wr_pad, br_pad):
    return pl.pallas_call(
        _final_body,
        grid=(GRID,),
        in_specs=[
            _row_spec(), _row_spec(), _row_spec(), _row_spec(),
            _full_spec((1, H)), _full_spec((H, H)), _full_spec((1, H)),
        ],
        out_specs=_row_spec(),
        out_shape=jax.ShapeDtypeStruct((NPAD, H), jnp.float32),
    )(a0, a1, zs, dinv2d, b, wr_pad, br_pad)


def kernel(x, edge_index, W1, b1, W2, b2, W3, b3, Wr, br):
    f32 = jnp.float32
    xp = jnp.zeros((NPAD, D), f32).at[:N].set(x)
    pk = jnp.full((EPAD + PKB * CHUNK,), N | (N << 16), jnp.int32)
    pk = pk.at[:E].set(edge_index[0] | (edge_index[1] << 16)).reshape(-1, CHUNK)
    ones_mat = jnp.ones((NPAD, H), f32)
    zeros_mat = jnp.zeros((NPAD, H), f32)
    wr_pad = jnp.zeros((H, H), f32).at[:, :OUT].set(Wr)
    br_pad = jnp.full((1, H), -1e30, f32).at[0, :OUT].set(br)

    d2 = _sc_degree(pk, ones_mat, zeros_mat)
    zs1, dinv2d = _tc_layer1(xp, d2[0], d2[1], W1)
    a1 = _sc_aggregate(zs1, pk)
    zs2 = _tc_combine(a1[0], a1[1], zs1, dinv2d, b1.reshape(1, H), W2)
    a2 = _sc_aggregate(zs2, pk)
    zs3 = _tc_combine(a2[0], a2[1], zs2, dinv2d, b2.reshape(1, H), W3)
    a3 = _sc_aggregate(zs3, pk)
    probs = _tc_final(a3[0], a3[1], zs3, dinv2d, b3.reshape(1, H), wr_pad, br_pad)
    return probs[:N, :OUT]
